# baseline (speedup 1.0000x reference)
import functools

import jax
import jax.numpy as jnp
from jax.experimental import pallas as pl
from jax.experimental.pallas import tpu as pltpu
from jax.experimental.pallas import tpu_sc as plsc


def _tc_project_fused(tok_fused, w2, b2):
    half, width = tok_fused.shape
    blk = 10000
    assert half % blk == 0

    def proj_kernel(f_ref, w_ref, b_ref, o_ref):
        o_ref[...] = (
            jnp.dot(f_ref[...], w_ref[...], preferred_element_type=jnp.float32)
            + b_ref[...]
        )

    return pl.pallas_call(
        proj_kernel,
        grid=(half // blk,),
        in_specs=[
            pl.BlockSpec((blk, width), lambda i: (i, 0)),
            pl.BlockSpec((width, width), lambda i: (0, 0)),
            pl.BlockSpec((1, width), lambda i: (0, 0)),
        ],
        out_specs=pl.BlockSpec((blk, width), lambda i: (i, 0)),
        out_shape=jax.ShapeDtypeStruct((half, width), jnp.float32),
    )(tok_fused, w2, b2)


def _sc_gather(table_fused, idx_fused):
    n = idx_fused.shape[0]
    width = table_fused.shape[1]
    window = 256
    assert n % window == 0
    mesh = plsc.VectorSubcoreMesh(core_axis_name="core", subcore_axis_name="subcore")
    idx2d = idx_fused.reshape(1, n)

    @functools.partial(
        pl.kernel,
        out_type=jax.ShapeDtypeStruct((n, width), table_fused.dtype),
        mesh=mesh,
    )
    def gather_kernel(tab_hbm, i_hbm, o_hbm):
        def body(i_vmem, o_vmem):
            pltpu.sync_copy(tab_hbm.at[i_vmem.at[0]], o_vmem)

        pltpu.emit_pipeline(
            body,
            grid=(n // window,),
            in_specs=[pl.BlockSpec((1, window), lambda i: (0, i))],
            out_specs=[pl.BlockSpec((window, width), lambda i: (i, 0))],
            core_axis_name=("core", "subcore"),
            dimension_semantics=(pltpu.PARALLEL,),
        )(i_hbm, o_hbm)

    return gather_kernel(table_fused, idx2d)


def kernel(x, tok_embs, W, b):
    batch, seqlen = x.shape
    vocab, dim = tok_embs.shape
    idx = x.reshape(-1)
    tok_fused = tok_embs.reshape(vocab // 2, 2 * dim)
    zero = jnp.zeros((dim, dim), jnp.float32)
    w2 = jnp.block([[W.T, zero], [zero, W.T]])
    b2 = jnp.concatenate([b, b]).reshape(1, 2 * dim)
    proj = _tc_project_fused(tok_fused, w2, b2)
    rows = _sc_gather(proj, idx >> 1)
    z = jnp.where((idx & 1 == 1)[:, None], rows[:, dim:], rows[:, :dim])
    return z.reshape(batch, seqlen, dim)


# speedup vs baseline: 2.1919x; 2.1919x over previous
import functools

import jax
import jax.numpy as jnp
from jax.experimental import pallas as pl
from jax.experimental.pallas import tpu as pltpu
from jax.experimental.pallas import tpu_sc as plsc


def _tc_project_table_t(embs_t, w, b):
    dim, vocab = embs_t.shape
    blk = 8192
    nsteps = -(-vocab // blk)

    def proj_kernel(e_ref, w_ref, b_ref, o_ref):
        z = (
            jax.lax.dot_general(
                e_ref[...],
                w_ref[...],
                (((0,), (1,)), ((), ())),
                preferred_element_type=jnp.float32,
            )
            + b_ref[...]
        )
        o_ref[:, :dim] = z
        o_ref[:, dim:] = jnp.zeros_like(z)

    return pl.pallas_call(
        proj_kernel,
        grid=(nsteps,),
        in_specs=[
            pl.BlockSpec((dim, blk), lambda i: (0, i)),
            pl.BlockSpec((dim, dim), lambda i: (0, 0)),
            pl.BlockSpec((1, dim), lambda i: (0, 0)),
        ],
        out_specs=pl.BlockSpec((blk, 2 * dim), lambda i: (i, 0)),
        out_shape=jax.ShapeDtypeStruct((vocab, 2 * dim), jnp.float32),
    )(embs_t, w, b.reshape(1, dim))


def _sc_gather(table_wide, idx_flat):
    n = idx_flat.shape[0]
    width = table_wide.shape[1]
    window = 256
    assert n % window == 0
    mesh = plsc.VectorSubcoreMesh(core_axis_name="core", subcore_axis_name="subcore")
    idx2d = idx_flat.reshape(1, n)

    @functools.partial(
        pl.kernel,
        out_type=jax.ShapeDtypeStruct((n, width), table_wide.dtype),
        mesh=mesh,
    )
    def gather_kernel(tab_hbm, i_hbm, o_hbm):
        def body(i_vmem, o_vmem):
            pltpu.sync_copy(tab_hbm.at[i_vmem.at[0]], o_vmem)

        pltpu.emit_pipeline(
            body,
            grid=(n // window,),
            in_specs=[pl.BlockSpec((1, window), lambda i: (0, i))],
            out_specs=[pl.BlockSpec((window, width), lambda i: (i, 0))],
            core_axis_name=("core", "subcore"),
            dimension_semantics=(pltpu.PARALLEL,),
        )(i_hbm, o_hbm)

    return gather_kernel(table_wide, idx2d)


def kernel(x, tok_embs, W, b):
    batch, seqlen = x.shape
    vocab, dim = tok_embs.shape
    proj = _tc_project_table_t(tok_embs.T, W, b)
    rows = _sc_gather(proj, x.reshape(-1))
    return rows[:, :dim].reshape(batch, seqlen, dim)


# speedup vs baseline: 2.2851x; 1.0426x over previous
import functools

import jax
import jax.numpy as jnp
from jax.experimental import pallas as pl
from jax.experimental.pallas import tpu as pltpu
from jax.experimental.pallas import tpu_sc as plsc


def _tc_project_table_t(embs_t, w, b):
    dim, vocab = embs_t.shape
    blk = 16384
    nsteps = -(-vocab // blk)

    def proj_kernel(e_ref, w_ref, b_ref, o_ref):
        z = (
            jax.lax.dot_general(
                e_ref[...],
                w_ref[...],
                (((0,), (1,)), ((), ())),
                preferred_element_type=jnp.float32,
            )
            + b_ref[...]
        )
        o_ref[:, :dim] = z
        o_ref[:, dim:] = jnp.zeros_like(z)

    return pl.pallas_call(
        proj_kernel,
        grid=(nsteps,),
        in_specs=[
            pl.BlockSpec((dim, blk), lambda i: (0, i)),
            pl.BlockSpec((dim, dim), lambda i: (0, 0)),
            pl.BlockSpec((1, dim), lambda i: (0, 0)),
        ],
        out_specs=pl.BlockSpec((blk, 2 * dim), lambda i: (i, 0)),
        out_shape=jax.ShapeDtypeStruct((vocab, 2 * dim), jnp.float32),
    )(embs_t, w, b.reshape(1, dim))


def _sc_gather(table_wide, idx_flat):
    n = idx_flat.shape[0]
    width = table_wide.shape[1]
    window = 256
    assert n % window == 0
    mesh = plsc.VectorSubcoreMesh(core_axis_name="core", subcore_axis_name="subcore")
    idx2d = idx_flat.reshape(1, n)

    @functools.partial(
        pl.kernel,
        out_type=jax.ShapeDtypeStruct((n, width), table_wide.dtype),
        mesh=mesh,
    )
    def gather_kernel(tab_hbm, i_hbm, o_hbm):
        def body(i_vmem, o_vmem):
            pltpu.sync_copy(tab_hbm.at[i_vmem.at[0]], o_vmem)

        pltpu.emit_pipeline(
            body,
            grid=(n // window,),
            in_specs=[pl.BlockSpec((1, window), lambda i: (0, i))],
            out_specs=[pl.BlockSpec((window, width), lambda i: (i, 0))],
            core_axis_name=("core", "subcore"),
            dimension_semantics=(pltpu.PARALLEL,),
        )(i_hbm, o_hbm)

    return gather_kernel(table_wide, idx2d)


def kernel(x, tok_embs, W, b):
    batch, seqlen = x.shape
    vocab, dim = tok_embs.shape
    proj = _tc_project_table_t(tok_embs.T, W, b)
    rows = _sc_gather(proj, x.reshape(-1))
    return rows[:, :dim].reshape(batch, seqlen, dim)
